# Initial kernel scaffold; baseline (speedup 1.0000x reference)
import jax
import jax.numpy as jnp
from jax.experimental import pallas as pl


def kernel(xyz, params):
    raise NotImplementedError("write your pallas kernel here")


# speedup vs baseline: 3.8446x; 3.8446x over previous
import functools

import jax
import jax.numpy as jnp
from jax import lax
from jax.experimental import pallas as pl


def _fps_body(xyz_ref, out_ref, *, npoint):
    xyz = xyz_ref[...]
    B, _, N = xyz.shape
    lanes = lax.broadcasted_iota(jnp.int32, (B, N), 1)
    out_lanes = lax.broadcasted_iota(jnp.int32, (B, 3, npoint), 2)

    def step(i, carry):
        dist_min, far, buf = carry
        oh = (lanes == far).astype(jnp.float32)
        cx = jnp.sum(xyz[:, 0, :] * oh, axis=1, keepdims=True)
        cy = jnp.sum(xyz[:, 1, :] * oh, axis=1, keepdims=True)
        cz = jnp.sum(xyz[:, 2, :] * oh, axis=1, keepdims=True)
        cent = jnp.concatenate(
            [cx[:, None, :], cy[:, None, :], cz[:, None, :]], axis=1
        )
        buf = jnp.where(out_lanes == i, cent, buf)
        tx = xyz[:, 0, :] - cx
        ty = xyz[:, 1, :] - cy
        tz = xyz[:, 2, :] - cz
        d = tx * tx + ty * ty + tz * tz
        dist_min = jnp.minimum(dist_min, d)
        m = jnp.max(dist_min, axis=1, keepdims=True)
        far = jnp.min(jnp.where(dist_min == m, lanes, N), axis=1, keepdims=True)
        return dist_min, far, buf

    init = (
        jnp.full((B, N), 1e10, jnp.float32),
        jnp.zeros((B, 1), jnp.int32),
        jnp.zeros((B, 3, npoint), jnp.float32),
    )
    _, _, buf = lax.fori_loop(0, npoint, step, init)
    out_ref[...] = buf


def _fps(xyz, npoint):
    B = xyz.shape[0]
    return pl.pallas_call(
        functools.partial(_fps_body, npoint=npoint),
        out_shape=jax.ShapeDtypeStruct((B, 3, npoint), jnp.float32),
    )(xyz)


def _prefix_sum(x):
    n = x.shape[-1]
    s = 1
    while s < n:
        pad = jnp.zeros(x.shape[:-1] + (s,), x.dtype)
        x = x + jnp.concatenate([pad, x[..., :-s]], axis=-1)
        s *= 2
    return x


def _group1_body(q_ref, xyz_ref, src_ref, wp_ref, wx_ref, b_ref, y_ref, st_ref,
                 *, radius2, K, nq, use_src):
    b = pl.program_id(0)
    i = pl.program_id(1)

    @pl.when(jnp.logical_and(b == 0, i == 0))
    def _():
        st_ref[...] = jnp.zeros_like(st_ref)

    ct = q_ref[0, 0]
    xyz = xyz_ref[0]
    N = xyz.shape[1]
    dn0 = (((0,), (0,)), ((), ()))
    dn = (((1,), (1,)), ((), ()))
    hi = jax.lax.Precision.HIGHEST
    ones3 = jnp.ones((3, 1), jnp.float32)
    q2 = lax.dot_general(ct * ct, ones3, dn0, precision=hi,
                         preferred_element_type=jnp.float32)
    x2 = jnp.sum(xyz * xyz, axis=0, keepdims=True)
    qx = lax.dot_general(
        ct.astype(jnp.bfloat16), xyz.astype(jnp.bfloat16), dn0,
        preferred_element_type=jnp.float32)
    d = q2 + x2 - 2.0 * qx
    mask = d <= radius2
    rank = _prefix_sum(mask.astype(jnp.float32))
    total = rank[:, N - 1 : N]
    jv = (lax.broadcasted_iota(jnp.int32, (1, K, 1), 1) + 1).astype(jnp.float32)
    r3 = rank[:, None, :]
    m3 = mask[:, None, :]
    t3 = total[:, :, None]
    lastcol = lax.broadcasted_iota(jnp.int32, (1, 1, N), 2) == N - 1
    sel = jnp.where(
        (m3 & ((r3 == jv) | ((jv > t3) & (r3 == 1.0))))
        | ((t3 == 0.0) & lastcol),
        1.0,
        0.0,
    )
    G = sel.reshape(nq * K, N)
    xg = lax.dot_general(xyz, G, dn, precision=hi,
                         preferred_element_type=jnp.float32)
    pidx = lax.broadcasted_iota(jnp.int32, (nq * K, nq), 0)
    qidx = lax.broadcasted_iota(jnp.int32, (nq * K, nq), 1)
    R = ((pidx >= qidx * K) & (pidx < (qidx + 1) * K)).astype(jnp.float32)
    crep = lax.dot_general(ct, R, dn, precision=hi,
                           preferred_element_type=jnp.float32)
    rel = xg - crep
    if use_src:
        feat = lax.dot_general(src_ref[0], G, dn, precision=hi,
                               preferred_element_type=jnp.float32)
    else:
        feat = xg
    y = (
        jnp.dot(wp_ref[...], feat, preferred_element_type=jnp.float32)
        + jnp.dot(wx_ref[...], rel, preferred_element_type=jnp.float32)
        + b_ref[...]
    )
    y_ref[0] = y
    st_ref[:, 0:1] += jnp.sum(y, axis=1, keepdims=True)
    st_ref[:, 1:2] += jnp.sum(y * y, axis=1, keepdims=True)


def _group_conv1(new_xyz, xyz, src, W1, b1, radius, K, TM=512):
    B, _, S = new_xyz.shape
    N = xyz.shape[2]
    nq = TM // K
    NB = (S * K) // TM
    c1 = W1.shape[0]
    use_src = src is not None
    if use_src:
        C = src.shape[1]
        wp = W1[:, :C]
        wx = W1[:, C:]
    else:
        C = 3
        src = xyz
        wp = W1[:, :3]
        wx = W1[:, 3:]
    q4 = jnp.transpose(new_xyz.reshape(B, 3, NB, nq), (0, 2, 1, 3))
    return pl.pallas_call(
        functools.partial(
            _group1_body, radius2=radius * radius, K=K, nq=nq, use_src=use_src
        ),
        grid=(B, NB),
        in_specs=[
            pl.BlockSpec((1, 1, 3, nq), lambda b, i: (b, i, 0, 0)),
            pl.BlockSpec((1, 3, N), lambda b, i: (b, 0, 0)),
            pl.BlockSpec((1, C, N), lambda b, i: (b, 0, 0)),
            pl.BlockSpec((c1, wp.shape[1]), lambda b, i: (0, 0)),
            pl.BlockSpec((c1, 3), lambda b, i: (0, 0)),
            pl.BlockSpec((c1, 1), lambda b, i: (0, 0)),
        ],
        out_specs=[
            pl.BlockSpec((1, c1, TM), lambda b, i: (b, 0, i)),
            pl.BlockSpec((c1, 8), lambda b, i: (0, 0)),
        ],
        out_shape=[
            jax.ShapeDtypeStruct((B, c1, S * K), jnp.float32),
            jax.ShapeDtypeStruct((c1, 8), jnp.float32),
        ],
    )(q4, xyz, src, wp, wx, b1.reshape(-1, 1))


def _bnconv_body(y_ref, st_ref, g_ref, be_ref, w_ref, b_ref, o_ref, st2_ref,
                 *, cnt):
    b = pl.program_id(0)
    i = pl.program_id(1)

    @pl.when(jnp.logical_and(b == 0, i == 0))
    def _():
        st2_ref[...] = jnp.zeros_like(st2_ref)

    mu = st_ref[:, 0:1] / cnt
    var = st_ref[:, 1:2] / cnt - mu * mu
    inv = lax.rsqrt(var + 1e-5)
    z = jnp.maximum((y_ref[0] - mu) * inv * g_ref[...] + be_ref[...], 0.0)
    o = jnp.dot(w_ref[...], z, preferred_element_type=jnp.float32) + b_ref[...]
    o_ref[0] = o
    st2_ref[:, 0:1] += jnp.sum(o, axis=1, keepdims=True)
    st2_ref[:, 1:2] += jnp.sum(o * o, axis=1, keepdims=True)


def _bn_conv(y, stats, g, be, W, bb, TM=2048):
    B, c, M = y.shape
    TM = min(TM, M)
    NB = M // TM
    co = W.shape[0]
    return pl.pallas_call(
        functools.partial(_bnconv_body, cnt=float(B * M)),
        grid=(B, NB),
        in_specs=[
            pl.BlockSpec((1, c, TM), lambda b, i: (b, 0, i)),
            pl.BlockSpec((c, 8), lambda b, i: (0, 0)),
            pl.BlockSpec((c, 1), lambda b, i: (0, 0)),
            pl.BlockSpec((c, 1), lambda b, i: (0, 0)),
            pl.BlockSpec((co, c), lambda b, i: (0, 0)),
            pl.BlockSpec((co, 1), lambda b, i: (0, 0)),
        ],
        out_specs=[
            pl.BlockSpec((1, co, TM), lambda b, i: (b, 0, i)),
            pl.BlockSpec((co, 8), lambda b, i: (0, 0)),
        ],
        out_shape=[
            jax.ShapeDtypeStruct((B, co, M), jnp.float32),
            jax.ShapeDtypeStruct((co, 8), jnp.float32),
        ],
    )(y, stats, g.reshape(-1, 1), be.reshape(-1, 1), W, bb.reshape(-1, 1))


def _bnmax_body(y_ref, st_ref, g_ref, be_ref, o_ref, *, cnt, K):
    mu = st_ref[:, 0:1] / cnt
    inv = lax.rsqrt(st_ref[:, 1:2] / cnt - mu * mu + 1e-5)
    z = jnp.maximum((y_ref[0] - mu) * inv * g_ref[...] + be_ref[...], 0.0)
    TM = z.shape[1]
    outs = [
        jnp.max(z[:, q * K : (q + 1) * K], axis=1, keepdims=True)
        for q in range(TM // K)
    ]
    o_ref[0, 0] = jnp.concatenate(outs, axis=1)


def _bn_maxpool(y, stats, g, be, K, TM=2048):
    B, c, M = y.shape
    TM = min(TM, M)
    NB = M // TM
    nqq = TM // K
    out = pl.pallas_call(
        functools.partial(_bnmax_body, cnt=float(B * M), K=K),
        grid=(B, NB),
        in_specs=[
            pl.BlockSpec((1, c, TM), lambda b, i: (b, 0, i)),
            pl.BlockSpec((c, 8), lambda b, i: (0, 0)),
            pl.BlockSpec((c, 1), lambda b, i: (0, 0)),
            pl.BlockSpec((c, 1), lambda b, i: (0, 0)),
        ],
        out_specs=pl.BlockSpec((1, 1, c, nqq), lambda b, i: (b, i, 0, 0)),
        out_shape=jax.ShapeDtypeStruct((B, NB, c, nqq), jnp.float32),
    )(y, stats, g.reshape(-1, 1), be.reshape(-1, 1))
    return jnp.transpose(out, (0, 2, 1, 3)).reshape(B, c, M // K)


def _sa_branch(new_xyz, xyz, src, blocks, radius, K):
    W1 = blocks[0]["W"]
    y, st = _group_conv1(new_xyz, xyz, src, W1, blocks[0]["b"], radius, K)
    for j in range(1, len(blocks)):
        p_prev = blocks[j - 1]
        y, st2 = _bn_conv(
            y, st, p_prev["g"], p_prev["be"], blocks[j]["W"], blocks[j]["b"]
        )
        st = st2
    p_last = blocks[-1]
    return _bn_maxpool(y, st, p_last["g"], p_last["be"], K)


def _bn_relu_list(ys, g, be, cnt):
    s = jnp.zeros((ys[0].shape[0], 1), jnp.float32)
    for y in ys:
        s = s + jnp.sum(y, axis=1, keepdims=True)
    mu = s / cnt
    v = jnp.zeros_like(s)
    for y in ys:
        t = y - mu
        v = v + jnp.sum(t * t, axis=1, keepdims=True)
    inv = lax.rsqrt(v / cnt + 1e-5)
    return [jnp.maximum((y - mu) * inv * g + be, 0.0) for y in ys]


def _sa3fp3_body(xyz_ref, pts_ref, *refs, nsa, nfp):
    o_ref = refs[-1]
    B = xyz_ref.shape[0]
    C = pts_ref.shape[1]
    p = list(refs[:-1])
    sa = [
        dict(W=p[4 * j][...], b=p[4 * j + 1][...], g=p[4 * j + 2][...],
             be=p[4 * j + 3][...])
        for j in range(nsa)
    ]
    fp = [
        dict(W=p[4 * (nsa + j)][...], b=p[4 * (nsa + j) + 1][...],
             g=p[4 * (nsa + j) + 2][...], be=p[4 * (nsa + j) + 3][...])
        for j in range(nfp)
    ]
    S = xyz_ref.shape[2]
    cnt = float(B * S)
    ys = [
        jnp.dot(sa[0]["W"][:, :3], xyz_ref[b], preferred_element_type=jnp.float32)
        + jnp.dot(sa[0]["W"][:, 3:], pts_ref[b], preferred_element_type=jnp.float32)
        + sa[0]["b"]
        for b in range(B)
    ]
    ys = _bn_relu_list(ys, sa[0]["g"], sa[0]["be"], cnt)
    for j in range(1, nsa):
        ys = [
            jnp.dot(sa[j]["W"], y, preferred_element_type=jnp.float32) + sa[j]["b"]
            for y in ys
        ]
        ys = _bn_relu_list(ys, sa[j]["g"], sa[j]["be"], cnt)
    l3 = [jnp.max(y, axis=1, keepdims=True) for y in ys]
    c1 = fp[0]["W"].shape[0]
    ys = [
        jnp.dot(fp[0]["W"][:, :C], pts_ref[b], preferred_element_type=jnp.float32)
        + jnp.dot(fp[0]["W"][:, C:], l3[b], preferred_element_type=jnp.float32)
        + fp[0]["b"]
        for b in range(B)
    ]
    ys = _bn_relu_list(ys, fp[0]["g"], fp[0]["be"], cnt)
    for j in range(1, nfp):
        ys = [
            jnp.dot(fp[j]["W"], y, preferred_element_type=jnp.float32) + fp[j]["b"]
            for y in ys
        ]
        ys = _bn_relu_list(ys, fp[j]["g"], fp[j]["be"], cnt)
    for b in range(B):
        o_ref[b] = ys[b]


def _sa3_fp3(xyz2, pts2, sa_blocks, fp_blocks):
    B, _, S = xyz2.shape
    flat = []
    for blk in list(sa_blocks) + list(fp_blocks):
        flat += [
            blk["W"],
            blk["b"].reshape(-1, 1),
            blk["g"].reshape(-1, 1),
            blk["be"].reshape(-1, 1),
        ]
    co = fp_blocks[-1]["W"].shape[0]
    return pl.pallas_call(
        functools.partial(_sa3fp3_body, nsa=len(sa_blocks), nfp=len(fp_blocks)),
        out_shape=jax.ShapeDtypeStruct((B, co, S), jnp.float32),
    )(xyz2, pts2, *flat)


def _fp_body(x1_ref, x2_ref, p1_ref, p2_ref, *refs, nfp):
    o_ref = refs[-1]
    B = x1_ref.shape[0]
    N1 = x1_ref.shape[2]
    S2 = x2_ref.shape[2]
    C1 = p1_ref.shape[1]
    p = list(refs[:-1])
    fp = [
        dict(W=p[4 * j][...], b=p[4 * j + 1][...], g=p[4 * j + 2][...],
             be=p[4 * j + 3][...])
        for j in range(nfp)
    ]
    cnt = float(B * N1)
    dn0 = (((0,), (0,)), ((), ()))
    dn1 = (((1,), (1,)), ((), ()))
    lane = lax.broadcasted_iota(jnp.int32, (N1, S2), 1)
    co = fp[0]["W"].shape[0]
    s = jnp.zeros((co, 1), jnp.float32)
    for b in range(B):
        x1 = x1_ref[b]
        x2 = x2_ref[b]
        n2 = jnp.sum(x1 * x1, axis=0, keepdims=True)
        s2v = jnp.sum(x2 * x2, axis=0, keepdims=True)
        qx = lax.dot_general(
            x1.astype(jnp.bfloat16), x2.astype(jnp.bfloat16), dn0,
            preferred_element_type=jnp.float32)
        d = n2.T + s2v - 2.0 * qx
        wsum = jnp.zeros((N1, 1), jnp.float32)
        wmat = jnp.zeros((N1, S2), jnp.float32)
        dd = d
        for _ in range(3):
            m = jnp.min(dd, axis=1, keepdims=True)
            idx = jnp.min(jnp.where(dd == m, lane, S2), axis=1, keepdims=True)
            r = 1.0 / (m + 1e-8)
            hit = lane == idx
            wmat = wmat + jnp.where(hit, r, 0.0)
            wsum = wsum + r
            dd = jnp.where(hit, 1e30, dd)
        wmat = wmat / wsum
        interp = lax.dot_general(
            p2_ref[b], wmat, dn1, precision=jax.lax.Precision.HIGHEST,
            preferred_element_type=jnp.float32
        )
        y = (
            jnp.dot(fp[0]["W"][:, :C1], p1_ref[b],
                    preferred_element_type=jnp.float32)
            + jnp.dot(fp[0]["W"][:, C1:], interp,
                      preferred_element_type=jnp.float32)
            + fp[0]["b"]
        )
        o_ref[b] = y
        s = s + jnp.sum(y, axis=1, keepdims=True)
    for j in range(nfp):
        mu = s / cnt
        v = jnp.zeros_like(s)
        for b in range(B):
            t = o_ref[b] - mu
            v = v + jnp.sum(t * t, axis=1, keepdims=True)
        inv = lax.rsqrt(v / cnt + 1e-5)
        last = j == nfp - 1
        if not last:
            s = jnp.zeros((fp[j + 1]["W"].shape[0], 1), jnp.float32)
        for b in range(B):
            z = jnp.maximum(
                (o_ref[b] - mu) * inv * fp[j]["g"] + fp[j]["be"], 0.0
            )
            if last:
                o_ref[b] = z
            else:
                y = (
                    jnp.dot(fp[j + 1]["W"], z,
                            preferred_element_type=jnp.float32)
                    + fp[j + 1]["b"]
                )
                o_ref[b] = y
                s = s + jnp.sum(y, axis=1, keepdims=True)


def _fp(x1, x2, p1, p2, blocks):
    B, _, N1 = x1.shape
    flat = []
    for blk in blocks:
        flat += [
            blk["W"],
            blk["b"].reshape(-1, 1),
            blk["g"].reshape(-1, 1),
            blk["be"].reshape(-1, 1),
        ]
    co = blocks[-1]["W"].shape[0]
    return pl.pallas_call(
        functools.partial(_fp_body, nfp=len(blocks)),
        out_shape=jax.ShapeDtypeStruct((B, co, N1), jnp.float32),
    )(x1, x2, p1, p2, *flat)


def kernel(xyz, params):
    l0_xyz = xyz
    l0_points = xyz

    l1_xyz = _fps(l0_xyz, 512)
    outs = []
    for radius, K, blocks in zip(
        [0.1, 0.2, 0.4], [32, 64, 128], params["sa1"]
    ):
        outs.append(_sa_branch(l1_xyz, l0_xyz, None, blocks, radius, K))
    l1_points = jnp.concatenate(outs, axis=1)

    l2_xyz = _fps(l1_xyz, 128)
    outs = []
    for radius, K, blocks in zip([0.4, 0.8], [64, 128], params["sa2"]):
        outs.append(_sa_branch(l2_xyz, l1_xyz, l1_points, blocks, radius, K))
    l2_points = jnp.concatenate(outs, axis=1)

    l2_points = _sa3_fp3(l2_xyz, l2_points, params["sa3"], params["fp3"])

    l1_points = _fp(l1_xyz, l2_xyz, l1_points, l2_points, params["fp2"])
    l0_points = _fp(l0_xyz, l1_xyz, l0_points, l1_points, params["fp1"])
    return l0_points
